# baseline (speedup 1.0000x reference)
import functools

import jax
import jax.numpy as jnp
from jax import lax
from jax.experimental import pallas as pl
from jax.experimental.pallas import tpu as pltpu
from jax.experimental.pallas import tpu_sc as plsc

_K = 16
_NB = 5
_LA = 3
_NPC = 8
_UNITS = 32


@functools.lru_cache(maxsize=None)
def _build(N, V, D, S, B):
    info = plsc.get_sparse_core_info()
    NC, NS = info.num_cores, info.num_subcores
    NW = NC * NS
    pos_per_w = S // NW
    assert pos_per_w == _NPC * _K and B * _NPC == _UNITS
    nslice = _K * D // 16
    mesh = plsc.VectorSubcoreMesh(core_axis_name="c", subcore_axis_name="s")

    @functools.partial(
        pl.kernel,
        out_type=jax.ShapeDtypeStruct((N, D), jnp.float32),
        mesh=mesh,
        scratch_types=[
            pltpu.VMEM((B, _NPC * _K), jnp.int32),
            [pltpu.VMEM((_K, D), jnp.float32) for _ in range(_NB)],
            [pltpu.VMEM((_K, D), jnp.float32) for _ in range(2)],
            pltpu.SemaphoreType.DMA,
            pltpu.SemaphoreType.DMA,
            pltpu.SemaphoreType.DMA,
        ],
    )
    def emb(x_hbm, table_hbm, pos_hbm, out_hbm, idx_v, gbuf, pbuf, sg, ss, sp):
        wid = lax.axis_index("s") * NC + lax.axis_index("c")
        pos0 = wid * pos_per_w

        xd = [pltpu.async_copy(x_hbm.at[b, pl.ds(pos0, _NPC * _K)],
                               idx_v.at[b], sp) for b in range(B)]
        p0d = pltpu.async_copy(pos_hbm.at[pl.ds(pos0, _K)], pbuf[0], sp)

        def add_unit(g, p):
            @plsc.parallel_loop(0, nslice, unroll=2)
            def body(j):
                r = j >> 6
                col = pl.multiple_of((j & (D // 16 - 1)) << 4, 16)
                plsc.addupdate(g.at[r, pl.ds(col, 16)], p[r, pl.ds(col, 16)])

        def fire_gather(u):
            pc, b = divmod(u, B)
            return pltpu.async_copy(
                table_hbm.at[idx_v.at[b, pl.ds(pc * _K, _K)]],
                gbuf[u % _NB], sg)

        def fire_store(u):
            pc, b = divmod(u, B)
            row0 = b * S + pos0 + pc * _K
            return pltpu.async_copy(
                gbuf[u % _NB], out_hbm.at[pl.ds(row0, _K)], ss)

        def fire_pos(pc):
            return pltpu.async_copy(
                pos_hbm.at[pl.ds(pos0 + pc * _K, _K)], pbuf[pc % 2], sp)

        for d in xd:
            d.wait()
        p0d.wait()
        gd = {u: fire_gather(u) for u in range(_LA)}
        sd, pd = {}, {}
        for u in range(_UNITS):
            nxt = u + _LA
            if nxt < _UNITS:
                if nxt >= _NB:
                    sd[nxt - _NB].wait()
                if nxt % B == 0:
                    pd[nxt // B] = fire_pos(nxt // B)
                gd[nxt] = fire_gather(nxt)
            if u % B == 0 and u > 0:
                pd[u // B].wait()
            gd[u].wait()
            add_unit(gbuf[u % _NB], pbuf[(u // B) % 2])
            sd[u] = fire_store(u)
        for u in range(_UNITS - _NB, _UNITS):
            sd[u].wait()

    return emb


def kernel(x, table, pos):
    B, S = x.shape
    V, D = table.shape
    N = B * S
    out = _build(N, V, D, S, B)(x.astype(jnp.int32), table, pos)
    return out.reshape(B, S, D)


# speedup vs baseline: 1.1793x; 1.1793x over previous
import functools

import jax
import jax.numpy as jnp
from jax import lax
from jax.experimental import pallas as pl
from jax.experimental.pallas import tpu as pltpu
from jax.experimental.pallas import tpu_sc as plsc

_K = 16
_NB = 5
_LA = 2
_NPC = 8
_UNITS = 32


@functools.lru_cache(maxsize=None)
def _build(N, V, D, S, B):
    info = plsc.get_sparse_core_info()
    NC, NS = info.num_cores, info.num_subcores
    NW = NC * NS
    pos_per_w = S // NW
    assert pos_per_w == _NPC * _K and B * _NPC == _UNITS
    nslice = _K * D // 16
    mesh = plsc.VectorSubcoreMesh(core_axis_name="c", subcore_axis_name="s")

    @functools.partial(
        pl.kernel,
        out_type=jax.ShapeDtypeStruct((N, D), jnp.float32),
        mesh=mesh,
        scratch_types=[
            pltpu.VMEM((B, _NPC * _K), jnp.int32),
            [pltpu.VMEM((_K, D), jnp.float32) for _ in range(_NB)],
            [pltpu.VMEM((_K, D), jnp.float32) for _ in range(2)],
            pltpu.SemaphoreType.DMA,
            pltpu.SemaphoreType.DMA,
            pltpu.SemaphoreType.DMA,
        ],
    )
    def emb(x_hbm, table_hbm, pos_hbm, out_hbm, idx_v, gbuf, pbuf, sg, ss, sp):
        wid = lax.axis_index("s") * NC + lax.axis_index("c")
        pos0 = wid * pos_per_w

        xd = [pltpu.async_copy(x_hbm.at[b, pl.ds(pos0, _NPC * _K)],
                               idx_v.at[b], sp) for b in range(B)]
        p0d = pltpu.async_copy(pos_hbm.at[pl.ds(pos0, _K)], pbuf[0], sp)

        def add_unit(g, p):
            @plsc.parallel_loop(0, nslice, unroll=4)
            def body(j):
                r = j >> 6
                col = pl.multiple_of((j & (D // 16 - 1)) << 4, 16)
                plsc.addupdate(g.at[r, pl.ds(col, 16)], p[r, pl.ds(col, 16)])

        def fire_gather(u):
            pc, b = divmod(u, B)
            return pltpu.async_copy(
                table_hbm.at[idx_v.at[b, pl.ds(pc * _K, _K)]],
                gbuf[u % _NB], sg)

        def fire_store(u):
            pc, b = divmod(u, B)
            row0 = b * S + pos0 + pc * _K
            return pltpu.async_copy(
                gbuf[u % _NB], out_hbm.at[pl.ds(row0, _K)], ss)

        def fire_pos(pc):
            return pltpu.async_copy(
                pos_hbm.at[pl.ds(pos0 + pc * _K, _K)], pbuf[pc % 2], sp)

        for d in xd:
            d.wait()
        p0d.wait()
        gd = {u: fire_gather(u) for u in range(_LA)}
        sd, pd = {}, {}
        for u in range(_UNITS):
            nxt = u + _LA
            if nxt < _UNITS:
                if nxt >= _NB:
                    sd[nxt - _NB].wait()
                if nxt % B == 0:
                    pd[nxt // B] = fire_pos(nxt // B)
                gd[nxt] = fire_gather(nxt)
            if u % B == 0 and u > 0:
                pd[u // B].wait()
            gd[u].wait()
            add_unit(gbuf[u % _NB], pbuf[(u // B) % 2])
            sd[u] = fire_store(u)
        for u in range(_UNITS - _NB, _UNITS):
            sd[u].wait()

    return emb


def kernel(x, table, pos):
    B, S = x.shape
    V, D = table.shape
    N = B * S
    out = _build(N, V, D, S, B)(x.astype(jnp.int32), table, pos)
    return out.reshape(B, S, D)


# speedup vs baseline: 1.1893x; 1.0085x over previous
import functools

import jax
import jax.numpy as jnp
from jax import lax
from jax.experimental import pallas as pl
from jax.experimental.pallas import tpu as pltpu
from jax.experimental.pallas import tpu_sc as plsc

_K = 16
_NB = 5
_LA = 3
_NPC = 8
_UNITS = 32


@functools.lru_cache(maxsize=None)
def _build(N, V, D, S, B):
    info = plsc.get_sparse_core_info()
    NC, NS = info.num_cores, info.num_subcores
    NW = NC * NS
    pos_per_w = S // NW
    assert pos_per_w == _NPC * _K and B * _NPC == _UNITS
    nslice = _K * D // 16
    mesh = plsc.VectorSubcoreMesh(core_axis_name="c", subcore_axis_name="s")

    @functools.partial(
        pl.kernel,
        out_type=jax.ShapeDtypeStruct((N, D), jnp.float32),
        mesh=mesh,
        scratch_types=[
            pltpu.VMEM((B, _NPC * _K), jnp.int32),
            [pltpu.VMEM((_K, D), jnp.float32) for _ in range(_NB)],
            [pltpu.VMEM((_K, D), jnp.float32) for _ in range(2)],
            pltpu.SemaphoreType.DMA,
            pltpu.SemaphoreType.DMA,
            pltpu.SemaphoreType.DMA,
        ],
    )
    def emb(x_hbm, table_hbm, pos_hbm, out_hbm, idx_v, gbuf, pbuf, sg, ss, sp):
        wid = lax.axis_index("s") * NC + lax.axis_index("c")
        pos0 = wid * pos_per_w

        xd = [pltpu.async_copy(x_hbm.at[b, pl.ds(pos0, _NPC * _K)],
                               idx_v.at[b], sp) for b in range(B)]
        p0d = pltpu.async_copy(pos_hbm.at[pl.ds(pos0, _K)], pbuf[0], sp)

        def add_unit(g, p):
            @plsc.parallel_loop(0, nslice, unroll=4)
            def body(j):
                r = j >> 6
                col = pl.multiple_of((j & (D // 16 - 1)) << 4, 16)
                plsc.addupdate(g.at[r, pl.ds(col, 16)], p[r, pl.ds(col, 16)])

        def fire_gather(u):
            pc, b = divmod(u, B)
            return pltpu.async_copy(
                table_hbm.at[idx_v.at[b, pl.ds(pc * _K, _K)]],
                gbuf[u % _NB], sg)

        def fire_store(u):
            pc, b = divmod(u, B)
            row0 = b * S + pos0 + pc * _K
            return pltpu.async_copy(
                gbuf[u % _NB], out_hbm.at[pl.ds(row0, _K)], ss)

        def fire_pos(pc):
            return pltpu.async_copy(
                pos_hbm.at[pl.ds(pos0 + pc * _K, _K)], pbuf[pc % 2], sp)

        for d in xd:
            d.wait()
        p0d.wait()
        gd = {u: fire_gather(u) for u in range(_LA)}
        sd, pd = {}, {}
        for u in range(_UNITS):
            nxt = u + _LA
            if nxt < _UNITS:
                if nxt >= _NB:
                    sd[nxt - _NB].wait()
                if nxt % B == 0:
                    pd[nxt // B] = fire_pos(nxt // B)
                gd[nxt] = fire_gather(nxt)
            if u % B == 0 and u > 0:
                pd[u // B].wait()
            gd[u].wait()
            add_unit(gbuf[u % _NB], pbuf[(u // B) % 2])
            sd[u] = fire_store(u)
        for u in range(_UNITS - _NB, _UNITS):
            sd[u].wait()

    return emb


def kernel(x, table, pos):
    B, S = x.shape
    V, D = table.shape
    N = B * S
    out = _build(N, V, D, S, B)(x.astype(jnp.int32), table, pos)
    return out.reshape(B, S, D)
